# Initial kernel scaffold; baseline (speedup 1.0000x reference)
import functools

import jax
import jax.numpy as jnp
from jax import lax
from jax.experimental import pallas as pl
from jax.experimental.pallas import tpu as pltpu
from jax.experimental.pallas import tpu_sc as plsc

N = 10000
E = 320000
D = 128
NC = 2
NS = 16
NW = NC * NS
EPT = E // NW
K = 80
NBLK = EPT // K
NPAD = 10240
DP = D + 16
RPT = NPAD // NS


def _tc_transform(x_ref, w_ref, asv_ref, adv_ref, h_ref, asrc_ref, adst_ref):
    h = jnp.dot(x_ref[...], w_ref[...], preferred_element_type=jnp.float32)
    h_ref[...] = h
    asrc_ref[...] = jnp.sum(h * asv_ref[...], axis=1, keepdims=True)
    adst_ref[...] = jnp.sum(h * adv_ref[...], axis=1, keepdims=True)


def _sc_edge(h_hbm, edges_hbm, asrc_hbm, adst_hbm, out_hbm,
             asrc_l, adst_l, src_l, dst_l, exrow, gbuf, msgbuf, sh_out, gsem):
    c = lax.axis_index("c")
    s = lax.axis_index("s")
    w = c * NS + s

    pltpu.sync_copy(asrc_hbm, asrc_l)
    pltpu.sync_copy(adst_hbm, adst_l)
    pltpu.sync_copy(edges_hbm.at[0, w], src_l)
    pltpu.sync_copy(edges_hbm.at[1, w], dst_l)

    def _zero(t, carry):
        r = t // (DP // 16)
        k = t % (DP // 16)
        msgbuf[r, pl.ds(k * 16, 16)] = jnp.zeros((16,), jnp.float32)
        return carry

    lax.fori_loop(0, K * (DP // 16), _zero, 0)
    for b in range(RPT // K):
        pltpu.sync_copy(msgbuf, sh_out.at[pl.ds(s * RPT + b * K, K)])

    def _mx(i, carry):
        ms, md = carry
        ms = jnp.maximum(ms, asrc_l[pl.ds(i * 16, 16)])
        md = jnp.maximum(md, adst_l[pl.ds(i * 16, 16)])
        return ms, md

    init = jnp.full((16,), -3.0e38, jnp.float32)
    ms, md = lax.fori_loop(0, N // 16, _mx, (init, init))
    smax = jnp.max(ms) + jnp.max(md)
    cshift = jnp.where(smax >= 0.0, smax, 0.2 * smax)

    plsc.subcore_barrier()

    def _row(j, carry):
        def _exg(g, inner):
            sv = src_l[j, pl.ds(g * 16, 16)]
            dv = dst_l[j, pl.ds(g * 16, 16)]
            av = plsc.load_gather(asrc_l, [sv]) + plsc.load_gather(adst_l, [dv])
            av = jnp.where(av >= 0.0, av, 0.2 * av)
            exrow[pl.ds(g * 16, 16)] = jnp.exp(av - cshift)
            return inner

        lax.fori_loop(0, K // 16, _exg, 0)

        pltpu.async_copy(h_hbm.at[src_l.at[j]], gbuf, gsem).wait()

        lane = lax.iota(jnp.int32, 16)

        def _scale(e, inner):
            exb = plsc.load_gather(exrow, [jnp.full((16,), e, jnp.int32)])
            for k in range(D // 16):
                msgbuf[e, pl.ds(k * 16, 16)] = gbuf[e, pl.ds(k * 16, 16)] * exb
            msgbuf[e, pl.ds(D, 16)] = jnp.where(lane == 0, exb,
                                                jnp.zeros((16,), jnp.float32))
            return inner

        lax.fori_loop(0, K, _scale, 0)

        pltpu.sync_copy(msgbuf, sh_out.at[dst_l.at[j]], add=True)
        return carry

    lax.fori_loop(0, NBLK, _row, 0)

    plsc.subcore_barrier()
    pltpu.sync_copy(sh_out.at[pl.ds(s * RPT, RPT)],
                    out_hbm.at[c, pl.ds(s * RPT, RPT)])


def _tc_combine(part_ref, x_ref, bias_ref, o_ref):
    p = part_ref[0, :N, :D] + part_ref[1, :N, :D]
    den = part_ref[0, :N, D:D + 1] + part_ref[1, :N, D:D + 1]
    inv = 1.0 / (den + 1e-16)
    o_ref[...] = jnp.maximum(p * inv + bias_ref[...], 0.0) + x_ref[...]


@functools.lru_cache(maxsize=1)
def _sc_call():
    mesh = plsc.VectorSubcoreMesh(core_axis_name="c", subcore_axis_name="s",
                                  num_cores=NC, num_subcores=NS)
    return pl.kernel(
        _sc_edge,
        out_type=jax.ShapeDtypeStruct((NC, NPAD, DP), jnp.float32),
        mesh=mesh,
        scratch_types=[
            pltpu.VMEM((N,), jnp.float32),
            pltpu.VMEM((N,), jnp.float32),
            pltpu.VMEM((NBLK, K), jnp.int32),
            pltpu.VMEM((NBLK, K), jnp.int32),
            pltpu.VMEM((K,), jnp.float32),
            pltpu.VMEM((K, D), jnp.float32),
            pltpu.VMEM((K, DP), jnp.float32),
            pltpu.VMEM_SHARED((NPAD, DP), jnp.float32),
            pltpu.SemaphoreType.DMA,
        ],
    )


def kernel(x, edge_index, W, att_src, att_dst, bias):
    x = x.astype(jnp.float32)
    ei = edge_index.astype(jnp.int32).reshape(2, NW, NBLK, K)

    h, asrc, adst = pl.pallas_call(
        _tc_transform,
        out_shape=[
            jax.ShapeDtypeStruct((N, D), jnp.float32),
            jax.ShapeDtypeStruct((N, 1), jnp.float32),
            jax.ShapeDtypeStruct((N, 1), jnp.float32),
        ],
    )(x, W, att_src.reshape(1, D), att_dst.reshape(1, D))

    part = _sc_call()(h, ei, asrc.reshape(N), adst.reshape(N))

    out = pl.pallas_call(
        _tc_combine,
        out_shape=jax.ShapeDtypeStruct((N, D), jnp.float32),
    )(part, x, bias.reshape(1, D))
    return out


# speedup vs baseline: 17.7243x; 17.7243x over previous
import functools

import jax
import jax.numpy as jnp
from jax import lax
from jax.experimental import pallas as pl
from jax.experimental.pallas import tpu as pltpu
from jax.experimental.pallas import tpu_sc as plsc

N = 10000
E = 320000
D = 128
NC = 2
NS = 16
NW = NC * NS
EPT = E // NW
K = 80
NBLK = EPT // K
NPAD = 10240
DR = NPAD // D
RPT = NPAD // NS


def _tc_transform(x_ref, w_ref, asv_ref, adv_ref, h_ref, asrc_ref, adst_ref):
    h = jnp.dot(x_ref[...], w_ref[...], preferred_element_type=jnp.float32)
    h_ref[...] = h
    asrc_ref[...] = jnp.sum(h * asv_ref[...], axis=1, keepdims=True)
    adst_ref[...] = jnp.sum(h * adv_ref[...], axis=1, keepdims=True)


def _sc_edge(h_hbm, edges_hbm, asrc_hbm, adst_hbm, msg_hbm, den_hbm,
             asrc_l, adst_l, srcb, dstb, exrow, idbuf, den_l, gbuf,
             sh_msg, sh_den, gsem):
    c = lax.axis_index("c")
    s = lax.axis_index("s")
    w = c * NS + s

    pltpu.sync_copy(asrc_hbm, asrc_l)
    pltpu.sync_copy(adst_hbm, adst_l)

    lane = lax.iota(jnp.int32, 16)
    zeros16 = jnp.zeros((16,), jnp.float32)

    def _zero(t, carry):
        r = t // (D // 16)
        k = t % (D // 16)
        gbuf[r, pl.ds(k * 16, 16)] = zeros16
        den_l[r, pl.ds(k * 16, 16)] = zeros16
        return carry

    lax.fori_loop(0, K * (D // 16), _zero, 0)
    for g in range(K // 16):
        idbuf[0, pl.ds(g * 16, 16)] = lane + (g * 16)

    for b in range(RPT // K):
        pltpu.sync_copy(gbuf, sh_msg.at[pl.ds(s * RPT + b * K, K)])

    @pl.when(s == 0)
    def _():
        pltpu.sync_copy(gbuf, sh_den)

    def _mx(i, carry):
        ms, md = carry
        ms = jnp.maximum(ms, asrc_l[pl.ds(i * 16, 16)])
        md = jnp.maximum(md, adst_l[pl.ds(i * 16, 16)])
        return ms, md

    init = jnp.full((16,), -3.0e38, jnp.float32)
    ms, md = lax.fori_loop(0, N // 16, _mx, (init, init))

    smax = ms
    for off in (8, 4, 2, 1):
        exrow[pl.ds(0, 16)] = smax
        smax = jnp.maximum(smax, plsc.load_gather(exrow, [lane ^ off]))
    dmax = md
    for off in (8, 4, 2, 1):
        exrow[pl.ds(0, 16)] = dmax
        dmax = jnp.maximum(dmax, plsc.load_gather(exrow, [lane ^ off]))
    smax = smax + dmax
    cshift = jnp.where(smax >= 0.0, smax, 0.2 * smax)

    plsc.subcore_barrier()

    def _row(j, carry):
        pltpu.sync_copy(edges_hbm.at[0, w, j], srcb)
        pltpu.sync_copy(edges_hbm.at[1, w, j], dstb)
        gcopy = pltpu.async_copy(h_hbm.at[srcb.at[0]], gbuf, gsem)

        def _exg(g, inner):
            sv = srcb[0, pl.ds(g * 16, 16)]
            dv = dstb[0, pl.ds(g * 16, 16)]
            av = plsc.load_gather(asrc_l, [sv]) + plsc.load_gather(adst_l, [dv])
            av = jnp.where(av >= 0.0, av, 0.2 * av)
            ex = jnp.exp(av - cshift)
            exrow[pl.ds(g * 16, 16)] = ex
            plsc.addupdate_scatter(
                den_l, [lax.shift_right_logical(dv, 7), dv & 127], ex)
            return inner

        lax.fori_loop(0, K // 16, _exg, 0)
        gcopy.wait()

        def _scale(e, inner):
            exb = plsc.load_gather(exrow, [jnp.full((16,), e, jnp.int32)])
            for k in range(D // 16):
                gbuf[e, pl.ds(k * 16, 16)] = gbuf[e, pl.ds(k * 16, 16)] * exb
            return inner

        lax.fori_loop(0, K, _scale, 0)

        pltpu.sync_copy(gbuf, sh_msg.at[dstb.at[0]], add=True)
        return carry

    lax.fori_loop(0, NBLK, _row, 0)

    pltpu.sync_copy(den_l, sh_den.at[idbuf.at[0]], add=True)
    plsc.subcore_barrier()
    pltpu.sync_copy(sh_msg.at[pl.ds(s * RPT, RPT)],
                    msg_hbm.at[c, pl.ds(s * RPT, RPT)])

    @pl.when(s == 0)
    def _():
        pltpu.sync_copy(sh_den, den_hbm.at[c])


def _tc_combine(part_ref, den_ref, x_ref, bias_ref, o_ref):
    p = part_ref[0, :N, :] + part_ref[1, :N, :]
    den = den_ref[0, :N, :] + den_ref[1, :N, :]
    inv = 1.0 / (den + 1e-16)
    o_ref[...] = jnp.maximum(p * inv + bias_ref[...], 0.0) + x_ref[...]


@functools.lru_cache(maxsize=1)
def _sc_call():
    mesh = plsc.VectorSubcoreMesh(core_axis_name="c", subcore_axis_name="s",
                                  num_cores=NC, num_subcores=NS)
    return pl.kernel(
        _sc_edge,
        out_type=[
            jax.ShapeDtypeStruct((NC, NPAD, D), jnp.float32),
            jax.ShapeDtypeStruct((NC, DR, D), jnp.float32),
        ],
        mesh=mesh,
        compiler_params=pltpu.CompilerParams(needs_layout_passes=False),
        scratch_types=[
            pltpu.VMEM((N,), jnp.float32),
            pltpu.VMEM((N,), jnp.float32),
            pltpu.VMEM((1, K), jnp.int32),
            pltpu.VMEM((1, K), jnp.int32),
            pltpu.VMEM((K,), jnp.float32),
            pltpu.VMEM((1, K), jnp.int32),
            pltpu.VMEM((DR, D), jnp.float32),
            pltpu.VMEM((K, D), jnp.float32),
            pltpu.VMEM_SHARED((NPAD, D), jnp.float32),
            pltpu.VMEM_SHARED((DR, D), jnp.float32),
            pltpu.SemaphoreType.DMA,
        ],
    )


def kernel(x, edge_index, W, att_src, att_dst, bias):
    x = x.astype(jnp.float32)
    ei = edge_index.astype(jnp.int32).reshape(2, NW, NBLK, 1, K)

    h, asrc, adst = pl.pallas_call(
        _tc_transform,
        out_shape=[
            jax.ShapeDtypeStruct((N, D), jnp.float32),
            jax.ShapeDtypeStruct((N, 1), jnp.float32),
            jax.ShapeDtypeStruct((N, 1), jnp.float32),
        ],
    )(x, W, att_src.reshape(1, D), att_dst.reshape(1, D))

    part, den = _sc_call()(h, ei, asrc.reshape(N), adst.reshape(N))

    out = pl.pallas_call(
        _tc_combine,
        out_shape=jax.ShapeDtypeStruct((N, D), jnp.float32),
    )(part, den.reshape(NC, NPAD, 1), x, bias.reshape(1, D))
    return out


# speedup vs baseline: 27.2824x; 1.5393x over previous
import functools

import jax
import jax.numpy as jnp
from jax import lax
from jax.experimental import pallas as pl
from jax.experimental.pallas import tpu as pltpu
from jax.experimental.pallas import tpu_sc as plsc

N = 10000
E = 320000
D = 128
NC = 2
NS = 16
NW = NC * NS
EPT = E // NW
K = 80
NBLK = EPT // K
NPAD = 10240
DR = NPAD // D
RPT = NPAD // NS


def _tc_transform(x_ref, w_ref, asv_ref, adv_ref, h_ref, asrc_ref, adst_ref):
    h = jnp.dot(x_ref[...], w_ref[...], preferred_element_type=jnp.float32)
    h_ref[...] = h
    asrc_ref[...] = jnp.sum(h * asv_ref[...], axis=1, keepdims=True)
    adst_ref[...] = jnp.sum(h * adv_ref[...], axis=1, keepdims=True)


def _sc_scores(edges_hbm, asrc_hbm, adst_hbm, ex_hbm, den_hbm,
               asrc_l, adst_l, src_l, dst_l, ex_l, idbuf, den_l, sh_den):
    c = lax.axis_index("c")
    s = lax.axis_index("s")
    w = c * NS + s

    pltpu.sync_copy(asrc_hbm, asrc_l)
    pltpu.sync_copy(adst_hbm, adst_l)
    pltpu.sync_copy(edges_hbm.at[0, w], src_l)
    pltpu.sync_copy(edges_hbm.at[1, w], dst_l)

    lane = lax.iota(jnp.int32, 16)
    zeros16 = jnp.zeros((16,), jnp.float32)

    def _zero(t, carry):
        r = t // (D // 16)
        k = t % (D // 16)
        den_l[r, pl.ds(k * 16, 16)] = zeros16
        return carry

    lax.fori_loop(0, DR * (D // 16), _zero, 0)
    for g in range(K // 16):
        idbuf[0, pl.ds(g * 16, 16)] = lane + (g * 16)

    @pl.when(s == 0)
    def _():
        pltpu.sync_copy(den_l, sh_den)

    def _mx(i, carry):
        ms, md = carry
        ms = jnp.maximum(ms, asrc_l[pl.ds(i * 16, 16)])
        md = jnp.maximum(md, adst_l[pl.ds(i * 16, 16)])
        return ms, md

    init = jnp.full((16,), -3.0e38, jnp.float32)
    ms, md = lax.fori_loop(0, N // 16, _mx, (init, init))

    smax = ms
    for off in (8, 4, 2, 1):
        ex_l[0, pl.ds(0, 16)] = smax
        smax = jnp.maximum(
            smax, plsc.load_gather(ex_l, [jnp.zeros((16,), jnp.int32),
                                          lane ^ off]))
    dmax = md
    for off in (8, 4, 2, 1):
        ex_l[0, pl.ds(0, 16)] = dmax
        dmax = jnp.maximum(
            dmax, plsc.load_gather(ex_l, [jnp.zeros((16,), jnp.int32),
                                          lane ^ off]))
    smax = smax + dmax
    cshift = jnp.where(smax >= 0.0, smax, 0.2 * smax)

    plsc.subcore_barrier()

    def _row(j, carry):
        def _exg(g, inner):
            sv = src_l[j, pl.ds(g * 16, 16)]
            dv = dst_l[j, pl.ds(g * 16, 16)]
            av = plsc.load_gather(asrc_l, [sv]) + plsc.load_gather(adst_l, [dv])
            av = jnp.where(av >= 0.0, av, 0.2 * av)
            ex = jnp.exp(av - cshift)
            ex_l[j, pl.ds(g * 16, 16)] = ex
            plsc.addupdate_scatter(
                den_l, [lax.shift_right_logical(dv, 7), dv & 127], ex)
            return inner

        return lax.fori_loop(0, K // 16, _exg, carry)

    lax.fori_loop(0, NBLK, _row, 0)

    pltpu.sync_copy(ex_l, ex_hbm.at[w])

    pltpu.sync_copy(den_l, sh_den.at[idbuf.at[0]], add=True)
    plsc.subcore_barrier()

    @pl.when(s == 0)
    def _():
        pltpu.sync_copy(sh_den, den_hbm.at[c])


def _sc_aggregate(h_hbm, edges_hbm, ex_hbm, msg_hbm,
                  eb0, eb1, eb2, xb0, xb1, xb2, gb0, gb1, gb2, sh_msg,
                  es0, es1, es2, gs0, gs1, gs2, ss0, ss1, ss2):
    c = lax.axis_index("c")
    s = lax.axis_index("s")
    w = c * NS + s

    eb = (eb0, eb1, eb2)
    xb = (xb0, xb1, xb2)
    gb = (gb0, gb1, gb2)
    es = (es0, es1, es2)
    gs = (gs0, gs1, gs2)
    ss = (ss0, ss1, ss2)

    zeros16 = jnp.zeros((16,), jnp.float32)

    def _zero(t, carry):
        r = t // (D // 16)
        k = t % (D // 16)
        gb0[r, pl.ds(k * 16, 16)] = zeros16
        return carry

    lax.fori_loop(0, K * (D // 16), _zero, 0)
    for b in range(RPT // K):
        pltpu.sync_copy(gb0, sh_msg.at[pl.ds(s * RPT + b * K, K)])
    plsc.subcore_barrier()

    def fetch(t, sl):
        pltpu.async_copy(edges_hbm.at[w, t], eb[sl], es[sl])
        pltpu.async_copy(ex_hbm.at[w, t], xb[sl], es[sl])

    def drain_fetch(t, sl):
        pltpu.make_async_copy(edges_hbm.at[w, t], eb[sl], es[sl]).wait()
        pltpu.make_async_copy(ex_hbm.at[w, t], xb[sl], es[sl]).wait()

    def gather(sl):
        pltpu.async_copy(h_hbm.at[eb[sl].at[0]], gb[sl], gs[sl])

    def wait_gather(sl):
        pltpu.make_async_copy(h_hbm.at[eb[sl].at[0]], gb[sl], gs[sl]).wait()

    def scatter(sl):
        pltpu.async_copy(gb[sl], sh_msg.at[eb[sl].at[1]], ss[sl], add=True)

    def wait_scatter(sl):
        pltpu.make_async_copy(gb[sl], sh_msg.at[eb[sl].at[1]], ss[sl]).wait()

    def scale(sl):
        g = gb[sl]
        x = xb[sl]

        def _sc(e, inner):
            exb = plsc.load_gather(x, [jnp.full((16,), e, jnp.int32)])
            for k in range(D // 16):
                g[e, pl.ds(k * 16, 16)] = g[e, pl.ds(k * 16, 16)] * exb
            return inner

        lax.fori_loop(0, K, _sc, 0)

    fetch(0, 0)
    fetch(1, 1)
    fetch(2, 2)
    drain_fetch(0, 0)
    gather(0)
    drain_fetch(1, 1)
    gather(1)
    wait_gather(0)
    scale(0)
    pltpu.sync_copy(gb0, sh_msg.at[eb0.at[1]], add=True)
    drain_fetch(2, 2)
    gather(2)
    fetch(3, 0)
    wait_gather(1)
    scale(1)
    pltpu.sync_copy(gb1, sh_msg.at[eb1.at[1]], add=True)

    def body(i, carry):
        t0 = 3 * i + 2
        for k, (sl, sln, slf) in enumerate(((2, 0, 1), (0, 1, 2), (1, 2, 0))):
            t = t0 + k

            @pl.when(t + 1 <= NBLK - 1)
            def _():
                drain_fetch(t + 1, sln)
                gather(sln)

            wait_gather(sl)
            scale(sl)
            pltpu.sync_copy(gb[sl], sh_msg.at[eb[sl].at[1]], add=True)

            @pl.when(t + 2 <= NBLK - 1)
            def _():
                fetch(t + 2, slf)
        return carry

    lax.fori_loop(0, (NBLK - 2) // 3, body, 0)
    plsc.subcore_barrier()
    pltpu.sync_copy(sh_msg.at[pl.ds(s * RPT, RPT)],
                    msg_hbm.at[c, pl.ds(s * RPT, RPT)])


def _tc_combine(part_ref, den_ref, x_ref, bias_ref, o_ref):
    p = part_ref[0, :N, :] + part_ref[1, :N, :]
    den = den_ref[0, :N, :] + den_ref[1, :N, :]
    inv = 1.0 / (den + 1e-16)
    o_ref[...] = jnp.maximum(p * inv + bias_ref[...], 0.0) + x_ref[...]


@functools.lru_cache(maxsize=1)
def _sc_scores_call():
    mesh = plsc.VectorSubcoreMesh(core_axis_name="c", subcore_axis_name="s",
                                  num_cores=NC, num_subcores=NS)
    return pl.kernel(
        _sc_scores,
        out_type=[
            jax.ShapeDtypeStruct((NW, NBLK, K), jnp.float32),
            jax.ShapeDtypeStruct((NC, DR, D), jnp.float32),
        ],
        mesh=mesh,
        compiler_params=pltpu.CompilerParams(needs_layout_passes=False),
        scratch_types=[
            pltpu.VMEM((N,), jnp.float32),
            pltpu.VMEM((N,), jnp.float32),
            pltpu.VMEM((NBLK, K), jnp.int32),
            pltpu.VMEM((NBLK, K), jnp.int32),
            pltpu.VMEM((NBLK, K), jnp.float32),
            pltpu.VMEM((1, K), jnp.int32),
            pltpu.VMEM((DR, D), jnp.float32),
            pltpu.VMEM_SHARED((DR, D), jnp.float32),
        ],
    )


@functools.lru_cache(maxsize=1)
def _sc_aggregate_call():
    mesh = plsc.VectorSubcoreMesh(core_axis_name="c", subcore_axis_name="s",
                                  num_cores=NC, num_subcores=NS)
    return pl.kernel(
        _sc_aggregate,
        out_type=jax.ShapeDtypeStruct((NC, NPAD, D), jnp.float32),
        mesh=mesh,
        compiler_params=pltpu.CompilerParams(needs_layout_passes=False),
        scratch_types=(
            [pltpu.VMEM((2, K), jnp.int32)] * 3 +
            [pltpu.VMEM((K,), jnp.float32)] * 3 +
            [pltpu.VMEM((K, D), jnp.float32)] * 3 +
            [pltpu.VMEM_SHARED((NPAD, D), jnp.float32)] +
            [pltpu.SemaphoreType.DMA] * 9
        ),
    )


def kernel(x, edge_index, W, att_src, att_dst, bias):
    x = x.astype(jnp.float32)
    ei = edge_index.astype(jnp.int32)
    eia = ei.reshape(2, NW, NBLK, K)
    eib = jnp.transpose(eia, (1, 2, 0, 3))

    h, asrc, adst = pl.pallas_call(
        _tc_transform,
        out_shape=[
            jax.ShapeDtypeStruct((N, D), jnp.float32),
            jax.ShapeDtypeStruct((N, 1), jnp.float32),
            jax.ShapeDtypeStruct((N, 1), jnp.float32),
        ],
    )(x, W, att_src.reshape(1, D), att_dst.reshape(1, D))

    ex, den = _sc_scores_call()(eia, asrc.reshape(N), adst.reshape(N))
    part = _sc_aggregate_call()(h, eib, ex)

    out = pl.pallas_call(
        _tc_combine,
        out_shape=jax.ShapeDtypeStruct((N, D), jnp.float32),
    )(part, den.reshape(NC, NPAD, 1), x, bias.reshape(1, D))
    return out


# speedup vs baseline: 33.8703x; 1.2415x over previous
import functools

import jax
import jax.numpy as jnp
from jax import lax
from jax.experimental import pallas as pl
from jax.experimental.pallas import tpu as pltpu
from jax.experimental.pallas import tpu_sc as plsc

N = 10000
E = 320000
D = 128
NC = 2
NS = 16
NW = NC * NS
EPT = E // NW
K = 80
NBLK = EPT // K
NPAD = 10240
DR = NPAD // D
RPT = NPAD // NS


def _tc_transform(x_ref, w_ref, asv_ref, adv_ref, h_ref, asrc_ref, adst_ref):
    h = jnp.dot(x_ref[...], w_ref[...], preferred_element_type=jnp.float32)
    h_ref[...] = h
    asrc_ref[...] = jnp.sum(h * asv_ref[...], axis=1, keepdims=True)
    adst_ref[...] = jnp.sum(h * adv_ref[...], axis=1, keepdims=True)


def _sc_scores(edges_hbm, asrc_hbm, adst_hbm, ex_hbm, den_hbm,
               asrc_l, adst_l, src_l, dst_l, ex_l, idbuf, den_l, sh_den):
    c = lax.axis_index("c")
    s = lax.axis_index("s")
    w = c * NS + s

    pltpu.sync_copy(asrc_hbm, asrc_l)
    pltpu.sync_copy(adst_hbm, adst_l)
    pltpu.sync_copy(edges_hbm.at[0, w], src_l)
    pltpu.sync_copy(edges_hbm.at[1, w], dst_l)

    lane = lax.iota(jnp.int32, 16)
    zeros16 = jnp.zeros((16,), jnp.float32)

    def _zero(t, carry):
        r = t // (D // 16)
        k = t % (D // 16)
        den_l[r, pl.ds(k * 16, 16)] = zeros16
        return carry

    lax.fori_loop(0, DR * (D // 16), _zero, 0)
    for g in range(K // 16):
        idbuf[0, pl.ds(g * 16, 16)] = lane + (g * 16)

    @pl.when(s == 0)
    def _():
        pltpu.sync_copy(den_l, sh_den)

    def _mx(i, carry):
        ms, md = carry
        ms = jnp.maximum(ms, asrc_l[pl.ds(i * 16, 16)])
        md = jnp.maximum(md, adst_l[pl.ds(i * 16, 16)])
        return ms, md

    init = jnp.full((16,), -3.0e38, jnp.float32)
    ms, md = lax.fori_loop(0, N // 16, _mx, (init, init))

    zi = jnp.zeros((16,), jnp.int32)
    smax = ms
    for off in (8, 4, 2, 1):
        ex_l[0, pl.ds(0, 16)] = smax
        smax = jnp.maximum(smax, plsc.load_gather(ex_l, [zi, lane ^ off]))
    dmax = md
    for off in (8, 4, 2, 1):
        ex_l[0, pl.ds(0, 16)] = dmax
        dmax = jnp.maximum(dmax, plsc.load_gather(ex_l, [zi, lane ^ off]))
    smax = smax + dmax
    cshift = jnp.where(smax >= 0.0, smax, 0.2 * smax)

    plsc.subcore_barrier()

    def _row(j, carry):
        def _exg(g, inner):
            sv = src_l[j, pl.ds(g * 16, 16)]
            dv = dst_l[j, pl.ds(g * 16, 16)]
            av = plsc.load_gather(asrc_l, [sv]) + plsc.load_gather(adst_l, [dv])
            av = jnp.where(av >= 0.0, av, 0.2 * av)
            ex = jnp.exp(av - cshift)
            ex_l[j, pl.ds(g * 16, 16)] = ex
            plsc.addupdate_scatter(
                den_l, [lax.shift_right_logical(dv, 7), dv & 127], ex)
            return inner

        return lax.fori_loop(0, K // 16, _exg, carry)

    lax.fori_loop(0, NBLK, _row, 0)

    pltpu.sync_copy(ex_l, ex_hbm.at[w])

    pltpu.sync_copy(den_l, sh_den.at[idbuf.at[0]], add=True)
    plsc.subcore_barrier()

    @pl.when(s == 0)
    def _():
        pltpu.sync_copy(sh_den, den_hbm.at[c])


def _sc_aggregate(h_hbm, edges_hbm, ex_hbm, msg_hbm,
                  sb0, sb1, sb2, db0, db1, db2, xb0, xb1, xb2,
                  gb0, gb1, gb2, sh_msg,
                  es0, es1, es2, gs0, gs1, gs2, ss0, ss1, ss2):
    c = lax.axis_index("c")
    s = lax.axis_index("s")
    w = c * NS + s

    sb = (sb0, sb1, sb2)
    db = (db0, db1, db2)
    xb = (xb0, xb1, xb2)
    gb = (gb0, gb1, gb2)
    es = (es0, es1, es2)
    gs = (gs0, gs1, gs2)
    ss = (ss0, ss1, ss2)

    zeros16 = jnp.zeros((16,), jnp.float32)

    def _zero(t, carry):
        r = t // (D // 16)
        k = t % (D // 16)
        gb0[r, pl.ds(k * 16, 16)] = zeros16
        return carry

    lax.fori_loop(0, K * (D // 16), _zero, 0)
    for b in range(RPT // K):
        pltpu.sync_copy(gb0, sh_msg.at[pl.ds(s * RPT + b * K, K)])
    plsc.subcore_barrier()

    def fetch(t, sl):
        pltpu.async_copy(edges_hbm.at[0, w, t], sb[sl], es[sl])
        pltpu.async_copy(edges_hbm.at[1, w, t], db[sl], es[sl])
        pltpu.async_copy(ex_hbm.at[w, t], xb[sl], es[sl])

    def drain_fetch(t, sl):
        pltpu.make_async_copy(edges_hbm.at[0, w, t], sb[sl], es[sl]).wait()
        pltpu.make_async_copy(edges_hbm.at[1, w, t], db[sl], es[sl]).wait()
        pltpu.make_async_copy(ex_hbm.at[w, t], xb[sl], es[sl]).wait()

    def gather(sl):
        pltpu.async_copy(h_hbm.at[sb[sl].at[0]], gb[sl], gs[sl])

    def wait_gather(sl):
        pltpu.make_async_copy(h_hbm.at[sb[sl].at[0]], gb[sl], gs[sl]).wait()

    def scatter(sl):
        pltpu.async_copy(gb[sl], sh_msg.at[db[sl].at[0]], ss[sl], add=True)

    def wait_scatter(sl):
        pltpu.make_async_copy(gb[sl], sh_msg.at[db[sl].at[0]], ss[sl]).wait()

    def scale(sl):
        g = gb[sl]
        x = xb[sl]

        def _sc(i, inner):
            e0 = i * 4
            for u in range(4):
                e = e0 + u
                exb = plsc.load_gather(x, [jnp.full((16,), e, jnp.int32)])
                for k in range(D // 16):
                    g[e, pl.ds(k * 16, 16)] = g[e, pl.ds(k * 16, 16)] * exb
            return inner

        lax.fori_loop(0, K // 4, _sc, 0)

    fetch(0, 0)
    fetch(1, 1)
    fetch(2, 2)
    drain_fetch(0, 0)
    gather(0)
    drain_fetch(1, 1)
    gather(1)
    wait_gather(0)
    scale(0)
    scatter(0)
    drain_fetch(2, 2)
    gather(2)
    wait_scatter(0)
    fetch(3, 0)
    wait_gather(1)
    scale(1)
    scatter(1)

    def body(i, carry):
        t0 = 3 * i + 2
        for k, (sl, sln, slf) in enumerate(((2, 0, 1), (0, 1, 2), (1, 2, 0))):
            t = t0 + k

            @pl.when(t + 1 <= NBLK - 1)
            def _():
                drain_fetch(t + 1, sln)
                gather(sln)

            wait_gather(sl)
            scale(sl)
            wait_scatter(slf)
            scatter(sl)

            @pl.when(t + 2 <= NBLK - 1)
            def _():
                fetch(t + 2, slf)
        return carry

    lax.fori_loop(0, (NBLK - 2) // 3, body, 0)
    wait_scatter((NBLK - 1) % 3)
    plsc.subcore_barrier()
    pltpu.sync_copy(sh_msg.at[pl.ds(s * RPT, RPT)],
                    msg_hbm.at[c, pl.ds(s * RPT, RPT)])


def _tc_combine(part_ref, den_ref, x_ref, bias_ref, o_ref):
    p = part_ref[0, :N, :] + part_ref[1, :N, :]
    den = den_ref[0, :N, :] + den_ref[1, :N, :]
    inv = 1.0 / (den + 1e-16)
    o_ref[...] = jnp.maximum(p * inv + bias_ref[...], 0.0) + x_ref[...]


@functools.lru_cache(maxsize=1)
def _sc_scores_call():
    mesh = plsc.VectorSubcoreMesh(core_axis_name="c", subcore_axis_name="s",
                                  num_cores=NC, num_subcores=NS)
    return pl.kernel(
        _sc_scores,
        out_type=[
            jax.ShapeDtypeStruct((NW, NBLK, K), jnp.float32),
            jax.ShapeDtypeStruct((NC, DR, D), jnp.float32),
        ],
        mesh=mesh,
        compiler_params=pltpu.CompilerParams(needs_layout_passes=False),
        scratch_types=[
            pltpu.VMEM((N,), jnp.float32),
            pltpu.VMEM((N,), jnp.float32),
            pltpu.VMEM((NBLK, K), jnp.int32),
            pltpu.VMEM((NBLK, K), jnp.int32),
            pltpu.VMEM((NBLK, K), jnp.float32),
            pltpu.VMEM((1, K), jnp.int32),
            pltpu.VMEM((DR, D), jnp.float32),
            pltpu.VMEM_SHARED((DR, D), jnp.float32),
        ],
    )


@functools.lru_cache(maxsize=1)
def _sc_aggregate_call():
    mesh = plsc.VectorSubcoreMesh(core_axis_name="c", subcore_axis_name="s",
                                  num_cores=NC, num_subcores=NS)
    return pl.kernel(
        _sc_aggregate,
        out_type=jax.ShapeDtypeStruct((NC, NPAD, D), jnp.float32),
        mesh=mesh,
        compiler_params=pltpu.CompilerParams(needs_layout_passes=False),
        scratch_types=(
            [pltpu.VMEM((1, K), jnp.int32)] * 6 +
            [pltpu.VMEM((K,), jnp.float32)] * 3 +
            [pltpu.VMEM((K, D), jnp.float32)] * 3 +
            [pltpu.VMEM_SHARED((NPAD, D), jnp.float32)] +
            [pltpu.SemaphoreType.DMA] * 9
        ),
    )


def kernel(x, edge_index, W, att_src, att_dst, bias):
    x = x.astype(jnp.float32)
    ei = edge_index.astype(jnp.int32)
    eia = ei.reshape(2, NW, NBLK, K)
    eib = ei.reshape(2, NW, NBLK, 1, K)

    h, asrc, adst = pl.pallas_call(
        _tc_transform,
        out_shape=[
            jax.ShapeDtypeStruct((N, D), jnp.float32),
            jax.ShapeDtypeStruct((N, 1), jnp.float32),
            jax.ShapeDtypeStruct((N, 1), jnp.float32),
        ],
    )(x, W, att_src.reshape(1, D), att_dst.reshape(1, D))

    ex, den = _sc_scores_call()(eia, asrc.reshape(N), adst.reshape(N))
    part = _sc_aggregate_call()(h, eib, ex)

    out = pl.pallas_call(
        _tc_combine,
        out_shape=jax.ShapeDtypeStruct((N, D), jnp.float32),
    )(part, den.reshape(NC, NPAD, 1), x, bias.reshape(1, D))
    return out
